# Initial kernel scaffold; baseline (speedup 1.0000x reference)
import jax
import jax.numpy as jnp
from jax.experimental import pallas as pl


def kernel(inputs, edge_index, edge_attr, W0, b0, g0, be0, W1, b1, g1, be1, Wl, bl, Wr, br, We, att, Wres, bg, g2, be2, Wq, bq):
    raise NotImplementedError("write your pallas kernel here")


# speedup vs baseline: 1.6411x; 1.6411x over previous
import functools

import jax
import jax.numpy as jnp
from jax import lax
from jax.experimental import pallas as pl
from jax.experimental.pallas import tpu as pltpu
from jax.experimental.pallas import tpu_sc as plsc

N = 10000
E = 320000
D_IN = 128
HID = 128
OUT = 256
N_ACT = 14
NPAD = 10240

NC = 2
NS = 16
LN_EPS = 1e-5

BN = 1000
KA = 80
KC = 80
EPW_A = E // (NC * NS)
EPT_C = E // NS


def _ln(x, g, b):
    mu = jnp.mean(x, axis=-1, keepdims=True)
    var = jnp.mean((x - mu) * (x - mu), axis=-1, keepdims=True)
    return (x - mu) * jax.lax.rsqrt(var + LN_EPS) * g + b


def _t1_body(inp, w0t, b0, g0, be0, w1t, b1, g1, be1, wlt, bl, wrt, br,
             wrest, bg, xl_o, xr_o, hb_o):
    x = _ln(jax.nn.relu(inp[...] @ w0t[...] + b0[...]), g0[...], be0[...])
    x = _ln(jax.nn.relu(x @ w1t[...] + b1[...]), g1[...], be1[...])
    xl_o[...] = x @ wlt[...] + bl[...]
    xr_o[...] = x @ wrt[...] + br[...]
    hb_o[...] = x @ wrest[...] + bg[...]


def _t1(inputs, w0t, b0, g0, be0, w1t, b1, g1, be1, wlt, bl, wrt, br,
        wrest, bg):
    nblk = N // BN
    full = lambda i: (0, 0)
    row = lambda i: (i, 0)
    return pl.pallas_call(
        _t1_body,
        grid=(nblk,),
        in_specs=[
            pl.BlockSpec((BN, D_IN), row),
            pl.BlockSpec((D_IN, HID), full), pl.BlockSpec((1, HID), full),
            pl.BlockSpec((1, HID), full), pl.BlockSpec((1, HID), full),
            pl.BlockSpec((HID, HID), full), pl.BlockSpec((1, HID), full),
            pl.BlockSpec((1, HID), full), pl.BlockSpec((1, HID), full),
            pl.BlockSpec((HID, OUT), full), pl.BlockSpec((1, OUT), full),
            pl.BlockSpec((HID, OUT), full), pl.BlockSpec((1, OUT), full),
            pl.BlockSpec((HID, OUT), full), pl.BlockSpec((1, OUT), full),
        ],
        out_specs=[
            pl.BlockSpec((BN, OUT), row),
            pl.BlockSpec((BN, OUT), row),
            pl.BlockSpec((BN, OUT), row),
        ],
        out_shape=[
            jax.ShapeDtypeStruct((N, OUT), jnp.float32),
            jax.ShapeDtypeStruct((N, OUT), jnp.float32),
            jax.ShapeDtypeStruct((N, OUT), jnp.float32),
        ],
    )(inputs, w0t, b0, g0, be0, w1t, b1, g1, be1, wlt, bl, wrt, br,
      wrest, bg)


def _ka_body(xl_hbm, xr_hbm, src_hbm, dst_hbm, ea0_hbm, ea1_hbm, ea2_hbm,
             wea_hbm, att_hbm,
             ex_hbm, den_hbm,
             srcv, dstv, ea0v, ea1v, ea2v, rows_l, rows_r, exv, we_v, att_v,
             zb, den_sh, sem1, sem2):
    cid = lax.axis_index("c")
    sid = lax.axis_index("s")
    wid = cid * NS + sid

    pltpu.sync_copy(wea_hbm, we_v)
    pltpu.sync_copy(att_hbm, att_v)

    def _zb(i, _):
        zb[pl.ds(i * 16, 16)] = jnp.zeros((16,), jnp.float32)
        return 0
    lax.fori_loop(0, 640 // 16, _zb, 0)

    if True:
        pltpu.sync_copy(zb, den_sh.at[pl.ds(sid * 640, 640)])
        plsc.subcore_barrier()

        base0 = wid * EPW_A

        lane = jnp.arange(16, dtype=jnp.int32)

        def chunk(j, _):
            base = base0 + j * KA
            pltpu.sync_copy(src_hbm.at[pl.ds(base, KA)], srcv)
            pltpu.sync_copy(dst_hbm.at[pl.ds(base, KA)], dstv)
            pltpu.sync_copy(ea0_hbm.at[pl.ds(base, KA)], ea0v)
            pltpu.sync_copy(ea1_hbm.at[pl.ds(base, KA)], ea1v)
            pltpu.sync_copy(ea2_hbm.at[pl.ds(base, KA)], ea2v)
            cl = pltpu.async_copy(xl_hbm.at[srcv], rows_l, sem1)
            cr = pltpu.async_copy(xr_hbm.at[dstv], rows_r, sem2)
            cl.wait()
            cr.wait()

            def group(g, _):
                gb = g * 16
                gsl = pl.ds(gb, 16)
                a0vec = ea0v[gsl]
                a1vec = ea1v[gsl]
                a2vec = ea2v[gsl]
                erow = gb + lane

                def cb_loop(cb, logit):
                    csl = pl.ds(cb * 16, 16)
                    w0vec = we_v[0, csl]
                    w1vec = we_v[1, csl]
                    w2vec = we_v[2, csl]
                    attvec = att_v[0, csl]
                    cb16 = cb * 16
                    for cc in range(16):
                        cvec = jnp.full((16,), cb16 + cc, dtype=jnp.int32)
                        lv = plsc.load_gather(rows_l, [erow, cvec])
                        rv = plsc.load_gather(rows_r, [erow, cvec])
                        v = lv + rv + a0vec * w0vec[cc] \
                            + a1vec * w1vec[cc] + a2vec * w2vec[cc]
                        v = jnp.maximum(v, 0.2 * v)
                        logit = logit + v * attvec[cc]
                    return logit

                logits = lax.fori_loop(0, OUT // 16, cb_loop,
                                       jnp.zeros((16,), jnp.float32))
                exv[gsl] = jnp.exp(logits)
                return 0
            lax.fori_loop(0, KA // 16, group, 0)

            pltpu.sync_copy(exv, den_sh.at[dstv], add=True)
            pltpu.sync_copy(exv, ex_hbm.at[pl.ds(base, KA)])
            return 0

        lax.fori_loop(0, EPW_A // KA, chunk, 0)
        plsc.subcore_barrier()
        pltpu.sync_copy(den_sh.at[pl.ds(sid * 640, 640)],
                        den_hbm.at[cid, pl.ds(sid * 640, 640)])


def _ka(xl, xr, src, dst, ea0, ea1, ea2, wea, att):
    mesh = plsc.VectorSubcoreMesh(core_axis_name="c", subcore_axis_name="s")
    f = pl.kernel(
        _ka_body,
        compiler_params=pltpu.CompilerParams(use_tc_tiling_on_sc=False,
                                            needs_layout_passes=False),
        out_type=[
            jax.ShapeDtypeStruct((E,), jnp.float32),
            jax.ShapeDtypeStruct((NC, NPAD), jnp.float32),
        ],
        mesh=mesh,
        scratch_types=[
            pltpu.VMEM((KA,), jnp.int32),
            pltpu.VMEM((KA,), jnp.int32),
            pltpu.VMEM((KA,), jnp.float32),
            pltpu.VMEM((KA,), jnp.float32),
            pltpu.VMEM((KA,), jnp.float32),
            pltpu.VMEM((KA, OUT), jnp.float32),
            pltpu.VMEM((KA, OUT), jnp.float32),
            pltpu.VMEM((KA,), jnp.float32),
            pltpu.VMEM((3, OUT), jnp.float32),
            pltpu.VMEM((1, OUT), jnp.float32),
            pltpu.VMEM((640,), jnp.float32),
            pltpu.VMEM_SHARED((NPAD,), jnp.float32),
            pltpu.SemaphoreType.DMA,
            pltpu.SemaphoreType.DMA,
        ],
    )
    return f(xl, xr, src, dst, ea0, ea1, ea2, wea, att)


def _kc_body(xl2_hbm, src_hbm, dst_hbm, ex_hbm, den_hbm,
             agg_hbm,
             srcv, gidx, dstv, exv, alv, rows, den_v, tmp_v, agg_sh, sem1):
    cid = lax.axis_index("c")
    sid = lax.axis_index("s")

    pltpu.sync_copy(den_hbm.at[0], den_v)
    pltpu.sync_copy(den_hbm.at[1], tmp_v)

    def _dadd(i, _):
        sl = pl.ds(i * 16, 16)
        den_v[sl] = den_v[sl] + tmp_v[sl]
        return 0
    lax.fori_loop(0, NPAD // 16, _dadd, 0)

    def _zr(k, _):
        for c in range(128 // 16):
            rows[k, pl.ds(c * 16, 16)] = jnp.zeros((16,), jnp.float32)
        return 0
    lax.fori_loop(0, KC, _zr, 0)

    if True:
        for t in range(640 // KC):
            pltpu.sync_copy(rows, agg_sh.at[pl.ds(sid * 640 + t * KC, KC)])
        plsc.subcore_barrier()

        base0 = sid * EPT_C

        def chunk(j, _):
            base = base0 + j * KC
            pltpu.sync_copy(src_hbm.at[pl.ds(base, KC)], srcv)
            pltpu.sync_copy(dst_hbm.at[pl.ds(base, KC)], dstv)
            pltpu.sync_copy(ex_hbm.at[pl.ds(base, KC)], exv)
            for i in range(KC // 16):
                sl = pl.ds(i * 16, 16)
                gidx[sl] = srcv[sl] * 2 + cid
            cg = pltpu.async_copy(xl2_hbm.at[gidx], rows, sem1)
            cg.wait()
            for i in range(KC // 16):
                sl = pl.ds(i * 16, 16)
                dv = plsc.load_gather(den_v, [dstv[sl]])
                alv[sl] = exv[sl] / (dv + 1e-30)

            def scale(g, _):
                gb = g * 16
                avec = alv[pl.ds(gb, 16)]
                for k in range(16):
                    a = avec[k]
                    r = gb + k
                    for c in range(128 // 16):
                        cs = pl.ds(c * 16, 16)
                        rows[r, cs] = rows[r, cs] * a
                return 0
            lax.fori_loop(0, KC // 16, scale, 0)
            pltpu.sync_copy(rows, agg_sh.at[dstv], add=True)
            return 0

        lax.fori_loop(0, EPT_C // KC, chunk, 0)
        plsc.subcore_barrier()
        pltpu.sync_copy(agg_sh.at[pl.ds(sid * 640, 640)],
                        agg_hbm.at[cid, pl.ds(sid * 640, 640)])


def _kc(xl2, src, dst, ex, den):
    mesh = plsc.VectorSubcoreMesh(core_axis_name="c", subcore_axis_name="s")
    f = pl.kernel(
        _kc_body,
        compiler_params=pltpu.CompilerParams(use_tc_tiling_on_sc=False,
                                            needs_layout_passes=False),
        out_type=jax.ShapeDtypeStruct((NC, NPAD, 128), jnp.float32),
        mesh=mesh,
        scratch_types=[
            pltpu.VMEM((KC,), jnp.int32),
            pltpu.VMEM((KC,), jnp.int32),
            pltpu.VMEM((KC,), jnp.int32),
            pltpu.VMEM((KC,), jnp.float32),
            pltpu.VMEM((KC,), jnp.float32),
            pltpu.VMEM((KC, 128), jnp.float32),
            pltpu.VMEM((NPAD,), jnp.float32),
            pltpu.VMEM((NPAD,), jnp.float32),
            pltpu.VMEM_SHARED((NPAD, 128), jnp.float32),
            pltpu.SemaphoreType.DMA,
        ],
    )
    return f(xl2, src, dst, ex, den)


def _t2_body(agg0, agg1, hb, g2, be2, wqt, bq, q_o):
    h = jnp.concatenate([agg0[...], agg1[...]], axis=1) + hb[...]
    h = _ln(jax.nn.relu(h), g2[...], be2[...])
    q_o[...] = h @ wqt[...] + bq[...]


def _t2(agg0, agg1, hb, g2, be2, wqt, bq):
    nblk = N // BN
    full = lambda i: (0, 0)
    row = lambda i: (i, 0)
    return pl.pallas_call(
        _t2_body,
        grid=(nblk,),
        in_specs=[
            pl.BlockSpec((BN, 128), row),
            pl.BlockSpec((BN, 128), row),
            pl.BlockSpec((BN, OUT), row),
            pl.BlockSpec((1, OUT), full), pl.BlockSpec((1, OUT), full),
            pl.BlockSpec((OUT, N_ACT), full), pl.BlockSpec((1, N_ACT), full),
        ],
        out_specs=pl.BlockSpec((BN, N_ACT), row),
        out_shape=jax.ShapeDtypeStruct((N, N_ACT), jnp.float32),
    )(agg0, agg1, hb, g2, be2, wqt, bq)


def kernel(inputs, edge_index, edge_attr, W0, b0, g0, be0, W1, b1, g1, be1,
           Wl, bl, Wr, br, We, att, Wres, bg, g2, be2, Wq, bq):
    r1 = lambda v: v.reshape(1, -1)
    xl, xr, hb = _t1(
        inputs, W0.T, r1(b0), r1(g0), r1(be0), W1.T, r1(b1), r1(g1),
        r1(be1), Wl.T, r1(bl), Wr.T, r1(br), Wres.T, r1(bg))

    src = edge_index[0]
    dst = edge_index[1]
    ex, den = _ka(xl, xr, src, dst, edge_attr[:, 0], edge_attr[:, 1],
                  edge_attr[:, 2], We.T, r1(att))

    xl2 = xl.reshape(2 * N, 128)
    agg = _kc(xl2, src, dst, ex, den)

    q = _t2(agg[0, :N, :], agg[1, :N, :], hb, r1(g2), r1(be2), Wq.T, r1(bq))
    return q


# speedup vs baseline: 3.2719x; 1.9937x over previous
import functools

import jax
import jax.numpy as jnp
from jax import lax
from jax.experimental import pallas as pl
from jax.experimental.pallas import tpu as pltpu
from jax.experimental.pallas import tpu_sc as plsc

N = 10000
E = 320000
D_IN = 128
HID = 128
OUT = 256
N_ACT = 14
NPAD = 10240

NC = 2
NS = 16
LN_EPS = 1e-5

BN = 1000
KA = 80
KC = 80
EPW_A = E // (NC * NS)
EPT_C = E // NS


def _ln(x, g, b):
    mu = jnp.mean(x, axis=-1, keepdims=True)
    var = jnp.mean((x - mu) * (x - mu), axis=-1, keepdims=True)
    return (x - mu) * jax.lax.rsqrt(var + LN_EPS) * g + b


def _t1_body(inp, w0t, b0, g0, be0, w1t, b1, g1, be1, wlt, bl, wrt, br,
             wrest, bg, xl_o, xr_o, hb_o):
    x = _ln(jax.nn.relu(inp[...] @ w0t[...] + b0[...]), g0[...], be0[...])
    x = _ln(jax.nn.relu(x @ w1t[...] + b1[...]), g1[...], be1[...])
    xl_o[...] = x @ wlt[...] + bl[...]
    xr_o[...] = x @ wrt[...] + br[...]
    hb_o[...] = x @ wrest[...] + bg[...]


def _t1(inputs, w0t, b0, g0, be0, w1t, b1, g1, be1, wlt, bl, wrt, br,
        wrest, bg):
    nblk = N // BN
    full = lambda i: (0, 0)
    row = lambda i: (i, 0)
    return pl.pallas_call(
        _t1_body,
        grid=(nblk,),
        in_specs=[
            pl.BlockSpec((BN, D_IN), row),
            pl.BlockSpec((D_IN, HID), full), pl.BlockSpec((1, HID), full),
            pl.BlockSpec((1, HID), full), pl.BlockSpec((1, HID), full),
            pl.BlockSpec((HID, HID), full), pl.BlockSpec((1, HID), full),
            pl.BlockSpec((1, HID), full), pl.BlockSpec((1, HID), full),
            pl.BlockSpec((HID, OUT), full), pl.BlockSpec((1, OUT), full),
            pl.BlockSpec((HID, OUT), full), pl.BlockSpec((1, OUT), full),
            pl.BlockSpec((HID, OUT), full), pl.BlockSpec((1, OUT), full),
        ],
        out_specs=[
            pl.BlockSpec((BN, OUT), row),
            pl.BlockSpec((BN, OUT), row),
            pl.BlockSpec((BN, OUT), row),
        ],
        out_shape=[
            jax.ShapeDtypeStruct((N, OUT), jnp.float32),
            jax.ShapeDtypeStruct((N, OUT), jnp.float32),
            jax.ShapeDtypeStruct((N, OUT), jnp.float32),
        ],
    )(inputs, w0t, b0, g0, be0, w1t, b1, g1, be1, wlt, bl, wrt, br,
      wrest, bg)


def _ks_body(xl_hbm, xr_hbm, src_hbm, dst_hbm,
             s_hbm,
             srcv, dstv, rows_l, rows_r, sem1, sem2):
    cid = lax.axis_index("c")
    sid = lax.axis_index("s")
    wid = cid * NS + sid
    base0 = wid * EPW_A

    def chunk(j, _):
        base = base0 + j * KA
        pltpu.sync_copy(src_hbm.at[pl.ds(base, KA)], srcv)
        pltpu.sync_copy(dst_hbm.at[pl.ds(base, KA)], dstv)
        cl = pltpu.async_copy(xl_hbm.at[srcv], rows_l, sem1)
        cr = pltpu.async_copy(xr_hbm.at[dstv], rows_r, sem2)
        cl.wait()
        cr.wait()

        def row(k, _):
            for c in range(OUT // 16):
                sl = pl.ds(c * 16, 16)
                rows_l[k, sl] = rows_l[k, sl] + rows_r[k, sl]
            return 0
        lax.fori_loop(0, KA, row, 0)
        pltpu.sync_copy(rows_l, s_hbm.at[pl.ds(base, KA)])
        return 0

    lax.fori_loop(0, EPW_A // KA, chunk, 0)


def _ks(xl, xr, src, dst):
    mesh = plsc.VectorSubcoreMesh(core_axis_name="c", subcore_axis_name="s")
    f = pl.kernel(
        _ks_body,
        compiler_params=pltpu.CompilerParams(use_tc_tiling_on_sc=False,
                                            needs_layout_passes=False),
        out_type=jax.ShapeDtypeStruct((E, OUT), jnp.float32),
        mesh=mesh,
        scratch_types=[
            pltpu.VMEM((KA,), jnp.int32),
            pltpu.VMEM((KA,), jnp.int32),
            pltpu.VMEM((KA, OUT), jnp.float32),
            pltpu.VMEM((KA, OUT), jnp.float32),
            pltpu.SemaphoreType.DMA,
            pltpu.SemaphoreType.DMA,
        ],
    )
    return f(xl, xr, src, dst)


BE = 3200

def _tm_body(s_ref, ea3_ref, wet_ref, att_ref, ex_ref):
    ea = jax.lax.dot_general(ea3_ref[...], wet_ref[...],
                             (((0,), (0,)), ((), ())),
                             preferred_element_type=jnp.float32)
    v = s_ref[...] + ea
    v = jnp.maximum(v, 0.2 * v)
    ex_ref[...] = jnp.exp(jnp.sum(v * att_ref[...], axis=1, keepdims=True))


def _tm(s, ea3, wet, att):
    nblk = E // BE
    full = lambda i: (0, 0)
    return pl.pallas_call(
        _tm_body,
        grid=(nblk,),
        in_specs=[
            pl.BlockSpec((BE, OUT), lambda i: (i, 0)),
            pl.BlockSpec((3, BE), lambda i: (0, i)),
            pl.BlockSpec((3, OUT), full),
            pl.BlockSpec((1, OUT), full),
        ],
        out_specs=pl.BlockSpec((BE, 1), lambda i: (i, 0)),
        out_shape=jax.ShapeDtypeStruct((E, 1), jnp.float32),
    )(s, ea3, wet, att)


def _kc_body(xl2_hbm, src_hbm, dst_hbm, ex_hbm,
             agg_hbm,
             srcv, gidx, dstv, exv, alv, rows, den_v, zb,
             agg_sh, den_sh, sem1):
    cid = lax.axis_index("c")
    sid = lax.axis_index("s")

    def _zr(k, _):
        for c in range(128 // 16):
            rows[k, pl.ds(c * 16, 16)] = jnp.zeros((16,), jnp.float32)
        return 0
    lax.fori_loop(0, KC, _zr, 0)

    def _zb(i, _):
        zb[pl.ds(i * 16, 16)] = jnp.zeros((16,), jnp.float32)
        return 0
    lax.fori_loop(0, 640 // 16, _zb, 0)

    if True:
        for t in range(640 // KC):
            pltpu.sync_copy(rows, agg_sh.at[pl.ds(sid * 640 + t * KC, KC)])
        pltpu.sync_copy(zb, den_sh.at[pl.ds(sid * 640, 640)])
        plsc.subcore_barrier()

        base0 = sid * EPT_C

        def dchunk(j, _):
            base = base0 + j * KC
            pltpu.sync_copy(dst_hbm.at[pl.ds(base, KC)], dstv)
            pltpu.sync_copy(ex_hbm.at[pl.ds(base, KC)], exv)
            pltpu.sync_copy(exv, den_sh.at[dstv], add=True)
            return 0
        lax.fori_loop(0, EPT_C // KC, dchunk, 0)
        plsc.subcore_barrier()
        pltpu.sync_copy(den_sh, den_v)

        def chunk(j, _):
            base = base0 + j * KC
            pltpu.sync_copy(src_hbm.at[pl.ds(base, KC)], srcv)
            pltpu.sync_copy(dst_hbm.at[pl.ds(base, KC)], dstv)
            pltpu.sync_copy(ex_hbm.at[pl.ds(base, KC)], exv)
            for i in range(KC // 16):
                sl = pl.ds(i * 16, 16)
                gidx[sl] = srcv[sl] * 2 + cid
            cg = pltpu.async_copy(xl2_hbm.at[gidx], rows, sem1)
            cg.wait()
            for i in range(KC // 16):
                sl = pl.ds(i * 16, 16)
                dv = plsc.load_gather(den_v, [dstv[sl]])
                alv[sl] = exv[sl] / (dv + 1e-30)

            def scale(g, _):
                gb = g * 16
                avec = alv[pl.ds(gb, 16)]
                for k in range(16):
                    a = avec[k]
                    r = gb + k
                    for c in range(128 // 16):
                        cs = pl.ds(c * 16, 16)
                        rows[r, cs] = rows[r, cs] * a
                return 0
            lax.fori_loop(0, KC // 16, scale, 0)
            pltpu.sync_copy(rows, agg_sh.at[dstv], add=True)
            return 0

        lax.fori_loop(0, EPT_C // KC, chunk, 0)
        plsc.subcore_barrier()
        pltpu.sync_copy(agg_sh.at[pl.ds(sid * 640, 640)],
                        agg_hbm.at[cid, pl.ds(sid * 640, 640)])


def _kc(xl2, src, dst, ex):
    mesh = plsc.VectorSubcoreMesh(core_axis_name="c", subcore_axis_name="s")
    f = pl.kernel(
        _kc_body,
        compiler_params=pltpu.CompilerParams(use_tc_tiling_on_sc=False,
                                            needs_layout_passes=False),
        out_type=jax.ShapeDtypeStruct((NC, NPAD, 128), jnp.float32),
        mesh=mesh,
        scratch_types=[
            pltpu.VMEM((KC,), jnp.int32),
            pltpu.VMEM((KC,), jnp.int32),
            pltpu.VMEM((KC,), jnp.int32),
            pltpu.VMEM((KC,), jnp.float32),
            pltpu.VMEM((KC,), jnp.float32),
            pltpu.VMEM((KC, 128), jnp.float32),
            pltpu.VMEM((NPAD,), jnp.float32),
            pltpu.VMEM((640,), jnp.float32),
            pltpu.VMEM_SHARED((NPAD, 128), jnp.float32),
            pltpu.VMEM_SHARED((NPAD,), jnp.float32),
            pltpu.SemaphoreType.DMA,
        ],
    )
    return f(xl2, src, dst, ex)


def _t2_body(agg0, agg1, hb, g2, be2, wqt, bq, q_o):
    h = jnp.concatenate([agg0[...], agg1[...]], axis=1) + hb[...]
    h = _ln(jax.nn.relu(h), g2[...], be2[...])
    q_o[...] = h @ wqt[...] + bq[...]


def _t2(agg0, agg1, hb, g2, be2, wqt, bq):
    nblk = N // BN
    full = lambda i: (0, 0)
    row = lambda i: (i, 0)
    return pl.pallas_call(
        _t2_body,
        grid=(nblk,),
        in_specs=[
            pl.BlockSpec((BN, 128), row),
            pl.BlockSpec((BN, 128), row),
            pl.BlockSpec((BN, OUT), row),
            pl.BlockSpec((1, OUT), full), pl.BlockSpec((1, OUT), full),
            pl.BlockSpec((OUT, N_ACT), full), pl.BlockSpec((1, N_ACT), full),
        ],
        out_specs=pl.BlockSpec((BN, N_ACT), row),
        out_shape=jax.ShapeDtypeStruct((N, N_ACT), jnp.float32),
    )(agg0, agg1, hb, g2, be2, wqt, bq)


def kernel(inputs, edge_index, edge_attr, W0, b0, g0, be0, W1, b1, g1, be1,
           Wl, bl, Wr, br, We, att, Wres, bg, g2, be2, Wq, bq):
    r1 = lambda v: v.reshape(1, -1)
    xl, xr, hb = _t1(
        inputs, W0.T, r1(b0), r1(g0), r1(be0), W1.T, r1(b1), r1(g1),
        r1(be1), Wl.T, r1(bl), Wr.T, r1(br), Wres.T, r1(bg))

    src = edge_index[0]
    dst = edge_index[1]
    s = _ks(xl, xr, src, dst)
    ex = _tm(s, edge_attr.T, We.T, r1(att)).reshape(E)

    xl2 = xl.reshape(2 * N, 128)
    agg = _kc(xl2, src, dst, ex)

    q = _t2(agg[0, :N, :], agg[1, :N, :], hb, r1(g2), r1(be2), Wq.T, r1(bq))
    return q


# speedup vs baseline: 3.3121x; 1.0123x over previous
import functools

import jax
import jax.numpy as jnp
from jax import lax
from jax.experimental import pallas as pl
from jax.experimental.pallas import tpu as pltpu
from jax.experimental.pallas import tpu_sc as plsc

N = 10000
E = 320000
D_IN = 128
HID = 128
OUT = 256
N_ACT = 14
NPAD = 10240
E2 = 327680

NC = 2
NS = 16
LN_EPS = 1e-5

BN = 1000
KA = 80
KC = 80
EPW_A = E // (NC * NS)
EPT_C = E // NS


def _ln(x, g, b):
    mu = jnp.mean(x, axis=-1, keepdims=True)
    var = jnp.mean((x - mu) * (x - mu), axis=-1, keepdims=True)
    return (x - mu) * jax.lax.rsqrt(var + LN_EPS) * g + b


def _t1_body(inp, w0t, b0, g0, be0, w1t, b1, g1, be1, wlt, bl, wrt, br,
             wrest, bg, xl_o, xr_o, hb_o):
    x = _ln(jax.nn.relu(inp[...] @ w0t[...] + b0[...]), g0[...], be0[...])
    x = _ln(jax.nn.relu(x @ w1t[...] + b1[...]), g1[...], be1[...])
    xl_o[...] = x @ wlt[...] + bl[...]
    xr_o[...] = x @ wrt[...] + br[...]
    hb_o[...] = x @ wrest[...] + bg[...]


def _t1(inputs, w0t, b0, g0, be0, w1t, b1, g1, be1, wlt, bl, wrt, br,
        wrest, bg):
    nblk = N // BN
    full = lambda i: (0, 0)
    row = lambda i: (i, 0)
    return pl.pallas_call(
        _t1_body,
        grid=(nblk,),
        in_specs=[
            pl.BlockSpec((BN, D_IN), row),
            pl.BlockSpec((D_IN, HID), full), pl.BlockSpec((1, HID), full),
            pl.BlockSpec((1, HID), full), pl.BlockSpec((1, HID), full),
            pl.BlockSpec((HID, HID), full), pl.BlockSpec((1, HID), full),
            pl.BlockSpec((1, HID), full), pl.BlockSpec((1, HID), full),
            pl.BlockSpec((HID, OUT), full), pl.BlockSpec((1, OUT), full),
            pl.BlockSpec((HID, OUT), full), pl.BlockSpec((1, OUT), full),
            pl.BlockSpec((HID, OUT), full), pl.BlockSpec((1, OUT), full),
        ],
        out_specs=[
            pl.BlockSpec((BN, OUT), row),
            pl.BlockSpec((BN, OUT), row),
            pl.BlockSpec((BN, OUT), row),
        ],
        out_shape=[
            jax.ShapeDtypeStruct((N, OUT), jnp.float32),
            jax.ShapeDtypeStruct((N, OUT), jnp.float32),
            jax.ShapeDtypeStruct((N, OUT), jnp.float32),
        ],
    )(inputs, w0t, b0, g0, be0, w1t, b1, g1, be1, wlt, bl, wrt, br,
      wrest, bg)


def _ks_body(xl_hbm, xr_hbm, src_hbm, dst_hbm,
             s_hbm,
             srcv, dstv, rows_l, rows_r, sem1, sem2):
    cid = lax.axis_index("c")
    sid = lax.axis_index("s")
    wid = cid * NS + sid
    base0 = wid * EPW_A

    def chunk(j, _):
        base = base0 + j * KA
        pltpu.sync_copy(src_hbm.at[pl.ds(base, KA)], srcv)
        pltpu.sync_copy(dst_hbm.at[pl.ds(base, KA)], dstv)
        cl = pltpu.async_copy(xl_hbm.at[srcv], rows_l, sem1)
        cr = pltpu.async_copy(xr_hbm.at[dstv], rows_r, sem2)
        cl.wait()
        cr.wait()

        def row(k, _):
            for c in range(OUT // 16):
                sl = pl.ds(c * 16, 16)
                rows_l[k, sl] = rows_l[k, sl] + rows_r[k, sl]
            return 0
        lax.fori_loop(0, KA, row, 0)
        pltpu.sync_copy(rows_l, s_hbm.at[pl.ds(base, KA)])
        return 0

    lax.fori_loop(0, EPW_A // KA, chunk, 0)


def _ks(xl, xr, src, dst):
    mesh = plsc.VectorSubcoreMesh(core_axis_name="c", subcore_axis_name="s")
    f = pl.kernel(
        _ks_body,
        compiler_params=pltpu.CompilerParams(use_tc_tiling_on_sc=False,
                                            needs_layout_passes=False),
        out_type=jax.ShapeDtypeStruct((E2, OUT), jnp.float32),
        mesh=mesh,
        scratch_types=[
            pltpu.VMEM((KA,), jnp.int32),
            pltpu.VMEM((KA,), jnp.int32),
            pltpu.VMEM((KA, OUT), jnp.float32),
            pltpu.VMEM((KA, OUT), jnp.float32),
            pltpu.SemaphoreType.DMA,
            pltpu.SemaphoreType.DMA,
        ],
    )
    return f(xl, xr, src, dst)


BE = 2048

def _tm_body(s_ref, ea3_ref, wet_ref, att_ref, ex_ref):
    ea = jax.lax.dot_general(ea3_ref[...], wet_ref[...],
                             (((0,), (0,)), ((), ())),
                             preferred_element_type=jnp.float32)
    v = s_ref[...] + ea
    v = jnp.maximum(v, 0.2 * v)
    lg = jax.lax.dot_general(att_ref[...], v, (((1,), (1,)), ((), ())),
                             preferred_element_type=jnp.float32)
    ex_ref[...] = jnp.exp(lg[0])


def _tm(s, ea3, wet, att):
    nblk = E2 // BE
    full = lambda i: (0, 0)
    return pl.pallas_call(
        _tm_body,
        grid=(nblk,),
        in_specs=[
            pl.BlockSpec((BE, OUT), lambda i: (i, 0)),
            pl.BlockSpec((3, BE), lambda i: (0, i)),
            pl.BlockSpec((3, OUT), full),
            pl.BlockSpec((1, OUT), full),
        ],
        out_specs=pl.BlockSpec((BE,), lambda i: (i,)),
        out_shape=jax.ShapeDtypeStruct((E2,), jnp.float32),
    )(s, ea3, wet, att)


def _kc_body(xl2_hbm, src_hbm, dst_hbm, ex_hbm,
             agg_hbm,
             srcv, gidx, dstv, exv, alv, rows, den_v, zb,
             agg_sh, den_sh, sem1):
    cid = lax.axis_index("c")
    sid = lax.axis_index("s")

    def _zr(k, _):
        for c in range(128 // 16):
            rows[k, pl.ds(c * 16, 16)] = jnp.zeros((16,), jnp.float32)
        return 0
    lax.fori_loop(0, KC, _zr, 0)

    def _zb(i, _):
        zb[pl.ds(i * 16, 16)] = jnp.zeros((16,), jnp.float32)
        return 0
    lax.fori_loop(0, 640 // 16, _zb, 0)

    if True:
        for t in range(640 // KC):
            pltpu.sync_copy(rows, agg_sh.at[pl.ds(sid * 640 + t * KC, KC)])
        pltpu.sync_copy(zb, den_sh.at[pl.ds(sid * 640, 640)])
        plsc.subcore_barrier()

        base0 = sid * EPT_C

        def dchunk(j, _):
            base = base0 + j * KC
            pltpu.sync_copy(dst_hbm.at[pl.ds(base, KC)], dstv)
            pltpu.sync_copy(ex_hbm.at[pl.ds(base, KC)], exv)
            pltpu.sync_copy(exv, den_sh.at[dstv], add=True)
            return 0
        lax.fori_loop(0, EPT_C // KC, dchunk, 0)
        plsc.subcore_barrier()
        pltpu.sync_copy(den_sh, den_v)

        def chunk(j, _):
            base = base0 + j * KC
            pltpu.sync_copy(src_hbm.at[pl.ds(base, KC)], srcv)
            pltpu.sync_copy(dst_hbm.at[pl.ds(base, KC)], dstv)
            pltpu.sync_copy(ex_hbm.at[pl.ds(base, KC)], exv)
            for i in range(KC // 16):
                sl = pl.ds(i * 16, 16)
                gidx[sl] = srcv[sl] * 2 + cid
            cg = pltpu.async_copy(xl2_hbm.at[gidx], rows, sem1)
            cg.wait()
            for i in range(KC // 16):
                sl = pl.ds(i * 16, 16)
                dv = plsc.load_gather(den_v, [dstv[sl]])
                alv[sl] = exv[sl] / (dv + 1e-30)

            def scale(g, _):
                gb = g * 16
                avec = alv[pl.ds(gb, 16)]
                for k in range(16):
                    a = avec[k]
                    r = gb + k
                    for c in range(128 // 16):
                        cs = pl.ds(c * 16, 16)
                        rows[r, cs] = rows[r, cs] * a
                return 0
            lax.fori_loop(0, KC // 16, scale, 0)
            pltpu.sync_copy(rows, agg_sh.at[dstv], add=True)
            return 0

        lax.fori_loop(0, EPT_C // KC, chunk, 0)
        plsc.subcore_barrier()
        pltpu.sync_copy(agg_sh.at[pl.ds(sid * 640, 640)],
                        agg_hbm.at[cid, pl.ds(sid * 640, 640)])


def _kc(xl2, src, dst, ex):
    mesh = plsc.VectorSubcoreMesh(core_axis_name="c", subcore_axis_name="s")
    f = pl.kernel(
        _kc_body,
        compiler_params=pltpu.CompilerParams(use_tc_tiling_on_sc=False,
                                            needs_layout_passes=False),
        out_type=jax.ShapeDtypeStruct((NC, NPAD, 128), jnp.float32),
        mesh=mesh,
        scratch_types=[
            pltpu.VMEM((KC,), jnp.int32),
            pltpu.VMEM((KC,), jnp.int32),
            pltpu.VMEM((KC,), jnp.int32),
            pltpu.VMEM((KC,), jnp.float32),
            pltpu.VMEM((KC,), jnp.float32),
            pltpu.VMEM((KC, 128), jnp.float32),
            pltpu.VMEM((NPAD,), jnp.float32),
            pltpu.VMEM((640,), jnp.float32),
            pltpu.VMEM_SHARED((NPAD, 128), jnp.float32),
            pltpu.VMEM_SHARED((NPAD,), jnp.float32),
            pltpu.SemaphoreType.DMA,
        ],
    )
    return f(xl2, src, dst, ex)


def _t2_body(agg0, agg1, hb, g2, be2, wqt, bq, q_o):
    h = jnp.concatenate([agg0[...], agg1[...]], axis=1) + hb[...]
    h = _ln(jax.nn.relu(h), g2[...], be2[...])
    q_o[...] = h @ wqt[...] + bq[...]


def _t2(agg0, agg1, hb, g2, be2, wqt, bq):
    nblk = N // BN
    full = lambda i: (0, 0)
    row = lambda i: (i, 0)
    return pl.pallas_call(
        _t2_body,
        grid=(nblk,),
        in_specs=[
            pl.BlockSpec((BN, 128), row),
            pl.BlockSpec((BN, 128), row),
            pl.BlockSpec((BN, OUT), row),
            pl.BlockSpec((1, OUT), full), pl.BlockSpec((1, OUT), full),
            pl.BlockSpec((OUT, N_ACT), full), pl.BlockSpec((1, N_ACT), full),
        ],
        out_specs=pl.BlockSpec((BN, N_ACT), row),
        out_shape=jax.ShapeDtypeStruct((N, N_ACT), jnp.float32),
    )(agg0, agg1, hb, g2, be2, wqt, bq)


def kernel(inputs, edge_index, edge_attr, W0, b0, g0, be0, W1, b1, g1, be1,
           Wl, bl, Wr, br, We, att, Wres, bg, g2, be2, Wq, bq):
    r1 = lambda v: v.reshape(1, -1)
    xl, xr, hb = _t1(
        inputs, W0.T, r1(b0), r1(g0), r1(be0), W1.T, r1(b1), r1(g1),
        r1(be1), Wl.T, r1(bl), Wr.T, r1(br), Wres.T, r1(bg))

    src = edge_index[0]
    dst = edge_index[1]
    s = _ks(xl, xr, src, dst)
    ea3 = jnp.pad(edge_attr.T, ((0, 0), (0, E2 - E)))
    ex = _tm(s, ea3, We.T, r1(att))

    xl2 = xl.reshape(2 * N, 128)
    agg = _kc(xl2, src, dst, ex)

    q = _t2(agg[0, :N, :], agg[1, :N, :], hb, r1(g2), r1(be2), Wq.T, r1(bq))
    return q


# speedup vs baseline: 5.0033x; 1.5106x over previous
import functools

import jax
import jax.numpy as jnp
from jax import lax
from jax.experimental import pallas as pl
from jax.experimental.pallas import tpu as pltpu
from jax.experimental.pallas import tpu_sc as plsc

N = 10000
E = 320000
D_IN = 128
HID = 128
OUT = 256
N_ACT = 14
NPAD = 10240
E2 = 327680

NC = 2
NS = 16
LN_EPS = 1e-5

BN = 1000
KA = 80
KC = 80
EPW_A = E // (NC * NS)
EPT_C = E // NS


def _ln(x, g, b):
    mu = jnp.mean(x, axis=-1, keepdims=True)
    var = jnp.mean((x - mu) * (x - mu), axis=-1, keepdims=True)
    return (x - mu) * jax.lax.rsqrt(var + LN_EPS) * g + b


def _t1_body(inp, w0t, b0, g0, be0, w1t, b1, g1, be1, wlt, bl, wrt, br,
             wrest, bg, xl_o, xr_o, hb_o):
    x = _ln(jax.nn.relu(inp[...] @ w0t[...] + b0[...]), g0[...], be0[...])
    x = _ln(jax.nn.relu(x @ w1t[...] + b1[...]), g1[...], be1[...])
    xl_o[...] = x @ wlt[...] + bl[...]
    xr_o[...] = x @ wrt[...] + br[...]
    hb_o[...] = x @ wrest[...] + bg[...]


def _t1(inputs, w0t, b0, g0, be0, w1t, b1, g1, be1, wlt, bl, wrt, br,
        wrest, bg):
    nblk = N // BN
    full = lambda i: (0, 0)
    row = lambda i: (i, 0)
    return pl.pallas_call(
        _t1_body,
        grid=(nblk,),
        in_specs=[
            pl.BlockSpec((BN, D_IN), row),
            pl.BlockSpec((D_IN, HID), full), pl.BlockSpec((1, HID), full),
            pl.BlockSpec((1, HID), full), pl.BlockSpec((1, HID), full),
            pl.BlockSpec((HID, HID), full), pl.BlockSpec((1, HID), full),
            pl.BlockSpec((1, HID), full), pl.BlockSpec((1, HID), full),
            pl.BlockSpec((HID, OUT), full), pl.BlockSpec((1, OUT), full),
            pl.BlockSpec((HID, OUT), full), pl.BlockSpec((1, OUT), full),
            pl.BlockSpec((HID, OUT), full), pl.BlockSpec((1, OUT), full),
        ],
        out_specs=[
            pl.BlockSpec((BN, OUT), row),
            pl.BlockSpec((BN, OUT), row),
            pl.BlockSpec((BN, OUT), row),
        ],
        out_shape=[
            jax.ShapeDtypeStruct((N, OUT), jnp.float32),
            jax.ShapeDtypeStruct((N, OUT), jnp.float32),
            jax.ShapeDtypeStruct((N, OUT), jnp.float32),
        ],
    )(inputs, w0t, b0, g0, be0, w1t, b1, g1, be1, wlt, bl, wrt, br,
      wrest, bg)


def _ks_body(xl_hbm, xr_hbm, src_hbm, dst_hbm,
             s_hbm,
             srcv2, dstv2, rows_l2, rows_r2, sbuf2,
             gl0, gl1, gr0, gr1, w0, w1):
    cid = lax.axis_index("c")
    sid = lax.axis_index("s")
    wid = cid * NS + sid
    base0 = wid * EPW_A
    nch = EPW_A // KA
    gl = (gl0, gl1)
    gr = (gr0, gr1)
    ws = (w0, w1)

    def load_idx(jj, b):
        bs = base0 + jj * KA
        pltpu.sync_copy(src_hbm.at[pl.ds(bs, KA)], srcv2.at[b])
        pltpu.sync_copy(dst_hbm.at[pl.ds(bs, KA)], dstv2.at[b])

    def issue_g(b):
        pltpu.async_copy(xl_hbm.at[srcv2.at[b]], rows_l2.at[b], gl[b])
        pltpu.async_copy(xr_hbm.at[dstv2.at[b]], rows_r2.at[b], gr[b])

    for b in range(2):
        load_idx(b, b)
        issue_g(b)

    def stage(j, b):
        pltpu.make_async_copy(xl_hbm.at[srcv2.at[b]], rows_l2.at[b],
                              gl[b]).wait()
        pltpu.make_async_copy(xr_hbm.at[dstv2.at[b]], rows_r2.at[b],
                              gr[b]).wait()

        @pl.when(j >= 2)
        def _():
            pltpu.make_async_copy(
                sbuf2.at[b], s_hbm.at[pl.ds(base0 + (j - 2) * KA, KA)],
                ws[b]).wait()

        def row(k, _):
            for c in range(OUT // 16):
                sl = pl.ds(c * 16, 16)
                sbuf2[b, k, sl] = rows_l2[b, k, sl] + rows_r2[b, k, sl]
            return 0
        lax.fori_loop(0, KA, row, 0)
        pltpu.async_copy(sbuf2.at[b], s_hbm.at[pl.ds(base0 + j * KA, KA)],
                         ws[b])

        @pl.when(j + 2 < nch)
        def _():
            load_idx(j + 2, b)
            issue_g(b)

    def body(j, _):
        @pl.when(j % 2 == 0)
        def _():
            stage(j, 0)

        @pl.when(j % 2 == 1)
        def _():
            stage(j, 1)
        return 0
    lax.fori_loop(0, nch, body, 0)

    for b in range(2):
        jj = nch - 2 + b
        pltpu.make_async_copy(
            sbuf2.at[jj % 2], s_hbm.at[pl.ds(base0 + jj * KA, KA)],
            ws[jj % 2]).wait()


def _ks(xl, xr, src, dst):
    mesh = plsc.VectorSubcoreMesh(core_axis_name="c", subcore_axis_name="s")
    f = pl.kernel(
        _ks_body,
        compiler_params=pltpu.CompilerParams(use_tc_tiling_on_sc=False,
                                            needs_layout_passes=False),
        out_type=jax.ShapeDtypeStruct((E2, OUT), jnp.float32),
        mesh=mesh,
        scratch_types=[
            pltpu.VMEM((2, KA), jnp.int32),
            pltpu.VMEM((2, KA), jnp.int32),
            pltpu.VMEM((2, KA, OUT), jnp.float32),
            pltpu.VMEM((2, KA, OUT), jnp.float32),
            pltpu.VMEM((2, KA, OUT), jnp.float32),
            pltpu.SemaphoreType.DMA,
            pltpu.SemaphoreType.DMA,
            pltpu.SemaphoreType.DMA,
            pltpu.SemaphoreType.DMA,
            pltpu.SemaphoreType.DMA,
            pltpu.SemaphoreType.DMA,
        ],
    )
    return f(xl, xr, src, dst)


BE = 2048

def _tm_body(s_ref, ea3_ref, wet_ref, att_ref, ex_ref):
    ea = jax.lax.dot_general(ea3_ref[...], wet_ref[...],
                             (((0,), (0,)), ((), ())),
                             preferred_element_type=jnp.float32)
    v = s_ref[...] + ea
    v = jnp.maximum(v, 0.2 * v)
    lg = jax.lax.dot_general(att_ref[...], v, (((1,), (1,)), ((), ())),
                             preferred_element_type=jnp.float32)
    ex_ref[...] = jnp.exp(lg[0])


def _tm(s, ea3, wet, att):
    nblk = E2 // BE
    full = lambda i: (0, 0)
    return pl.pallas_call(
        _tm_body,
        grid=(nblk,),
        in_specs=[
            pl.BlockSpec((BE, OUT), lambda i: (i, 0)),
            pl.BlockSpec((3, BE), lambda i: (0, i)),
            pl.BlockSpec((3, OUT), full),
            pl.BlockSpec((1, OUT), full),
        ],
        out_specs=pl.BlockSpec((BE,), lambda i: (i,)),
        out_shape=jax.ShapeDtypeStruct((E2,), jnp.float32),
    )(s, ea3, wet, att)


def _kc_body(xl2_hbm, src_hbm, dst_hbm, ex_hbm,
             agg_hbm, den_hbm,
             srcv2, gidx2, dstv2, exv2, rows2, zb,
             agg_sh, den_sh, g0, g1):
    cid = lax.axis_index("c")
    sid = lax.axis_index("s")
    nch = EPT_C // KC
    gs = (g0, g1)

    def _zr(k, _):
        for c in range(128 // 16):
            rows2[0, k, pl.ds(c * 16, 16)] = jnp.zeros((16,), jnp.float32)
        return 0
    lax.fori_loop(0, KC, _zr, 0)

    def _zb(i, _):
        zb[pl.ds(i * 16, 16)] = jnp.zeros((16,), jnp.float32)
        return 0
    lax.fori_loop(0, 640 // 16, _zb, 0)

    for t in range(640 // KC):
        pltpu.sync_copy(rows2.at[0],
                        agg_sh.at[pl.ds(sid * 640 + t * KC, KC)])
    pltpu.sync_copy(zb, den_sh.at[pl.ds(sid * 640, 640)])
    plsc.subcore_barrier()

    base0 = sid * EPT_C

    def load_idx(jj, b):
        bs = base0 + jj * KC
        pltpu.sync_copy(src_hbm.at[pl.ds(bs, KC)], srcv2.at[b])
        pltpu.sync_copy(dst_hbm.at[pl.ds(bs, KC)], dstv2.at[b])
        pltpu.sync_copy(ex_hbm.at[pl.ds(bs, KC)], exv2.at[b])
        for i in range(KC // 16):
            sl = pl.ds(i * 16, 16)
            gidx2[b, sl] = srcv2[b, sl] * 2 + cid

    def issue_g(b):
        pltpu.async_copy(xl2_hbm.at[gidx2.at[b]], rows2.at[b], gs[b])

    for b in range(2):
        load_idx(b, b)
        issue_g(b)

    def stage(j, b):
        pltpu.make_async_copy(xl2_hbm.at[gidx2.at[b]], rows2.at[b],
                              gs[b]).wait()

        def scale(g, _):
            gb = g * 16
            evec = exv2[b, pl.ds(gb, 16)]
            for k in range(16):
                e = evec[k]
                r = gb + k
                for c in range(128 // 16):
                    cs = pl.ds(c * 16, 16)
                    rows2[b, r, cs] = rows2[b, r, cs] * e
            return 0
        lax.fori_loop(0, KC // 16, scale, 0)

        pltpu.sync_copy(rows2.at[b], agg_sh.at[dstv2.at[b]], add=True)
        pltpu.sync_copy(exv2.at[b], den_sh.at[dstv2.at[b]], add=True)

        @pl.when(j + 2 < nch)
        def _():
            load_idx(j + 2, b)
            issue_g(b)

    def body(j, _):
        @pl.when(j % 2 == 0)
        def _():
            stage(j, 0)

        @pl.when(j % 2 == 1)
        def _():
            stage(j, 1)
        return 0
    lax.fori_loop(0, nch, body, 0)

    plsc.subcore_barrier()
    pltpu.sync_copy(agg_sh.at[pl.ds(sid * 640, 640)],
                    agg_hbm.at[cid, pl.ds(sid * 640, 640)])
    pltpu.sync_copy(den_sh.at[pl.ds(sid * 640, 640)],
                    den_hbm.at[cid, pl.ds(sid * 640, 640)])


def _kc(xl2, src, dst, ex):
    mesh = plsc.VectorSubcoreMesh(core_axis_name="c", subcore_axis_name="s")
    f = pl.kernel(
        _kc_body,
        compiler_params=pltpu.CompilerParams(use_tc_tiling_on_sc=False,
                                            needs_layout_passes=False),
        out_type=[
            jax.ShapeDtypeStruct((NC, NPAD, 128), jnp.float32),
            jax.ShapeDtypeStruct((NC, NPAD), jnp.float32),
        ],
        mesh=mesh,
        scratch_types=[
            pltpu.VMEM((2, KC), jnp.int32),
            pltpu.VMEM((2, KC), jnp.int32),
            pltpu.VMEM((2, KC), jnp.int32),
            pltpu.VMEM((2, KC), jnp.float32),
            pltpu.VMEM((2, KC, 128), jnp.float32),
            pltpu.VMEM((640,), jnp.float32),
            pltpu.VMEM_SHARED((NPAD, 128), jnp.float32),
            pltpu.VMEM_SHARED((NPAD,), jnp.float32),
            pltpu.SemaphoreType.DMA,
            pltpu.SemaphoreType.DMA,
        ],
    )
    return f(xl2, src, dst, ex)


def _t2_body(agg0, agg1, den, hb, g2, be2, wqt, bq, q_o):
    inv = 1.0 / (den[...] + 1e-30)
    h = jnp.concatenate([agg0[...] * inv, agg1[...] * inv], axis=1) + hb[...]
    h = _ln(jax.nn.relu(h), g2[...], be2[...])
    q_o[...] = h @ wqt[...] + bq[...]


def _t2(agg0, agg1, den, hb, g2, be2, wqt, bq):
    nblk = N // BN
    full = lambda i: (0, 0)
    row = lambda i: (i, 0)
    return pl.pallas_call(
        _t2_body,
        grid=(nblk,),
        in_specs=[
            pl.BlockSpec((BN, 128), row),
            pl.BlockSpec((BN, 128), row),
            pl.BlockSpec((BN, 1), row),
            pl.BlockSpec((BN, OUT), row),
            pl.BlockSpec((1, OUT), full), pl.BlockSpec((1, OUT), full),
            pl.BlockSpec((OUT, N_ACT), full), pl.BlockSpec((1, N_ACT), full),
        ],
        out_specs=pl.BlockSpec((BN, N_ACT), row),
        out_shape=jax.ShapeDtypeStruct((N, N_ACT), jnp.float32),
    )(agg0, agg1, den, hb, g2, be2, wqt, bq)


def kernel(inputs, edge_index, edge_attr, W0, b0, g0, be0, W1, b1, g1, be1,
           Wl, bl, Wr, br, We, att, Wres, bg, g2, be2, Wq, bq):
    r1 = lambda v: v.reshape(1, -1)
    xl, xr, hb = _t1(
        inputs, W0.T, r1(b0), r1(g0), r1(be0), W1.T, r1(b1), r1(g1),
        r1(be1), Wl.T, r1(bl), Wr.T, r1(br), Wres.T, r1(bg))

    src = edge_index[0]
    dst = edge_index[1]
    s = _ks(xl, xr, src, dst)
    ea3 = jnp.pad(edge_attr.T, ((0, 0), (0, E2 - E)))
    ex = _tm(s, ea3, We.T, r1(att))

    xl2 = xl.reshape(2 * N, 128)
    agg, den = _kc(xl2, src, dst, ex)

    q = _t2(agg[0, :N, :], agg[1, :N, :], den[0, :N].reshape(N, 1), hb,
            r1(g2), r1(be2), Wq.T, r1(bq))
    return q
